# Initial kernel scaffold; baseline (speedup 1.0000x reference)
import jax
import jax.numpy as jnp
from jax.experimental import pallas as pl


def kernel(x, edge_index, edge_weight):
    raise NotImplementedError("write your pallas kernel here")


# speedup vs baseline: 11.0639x; 11.0639x over previous
import functools

import jax
import jax.numpy as jnp
from jax import lax
from jax.experimental import pallas as pl
from jax.experimental.pallas import tpu as pltpu
from jax.experimental.pallas import tpu_sc as plsc

NC = 2
NS = 16
L = 16
NW = NC * NS

N_NODES = 10000
N_EDGES = 320000
D = 128

EPW = N_EDGES // NW
K = 125
NCHUNK = EPW // K
RPT = N_NODES // NS
RPA = 624
RPL = N_NODES - (NS - 1) * RPA
CPD = D // L

_mesh = plsc.VectorSubcoreMesh(core_axis_name="c", subcore_axis_name="s")


def _softmax_stats_body(w_hbm, dst_hbm, ex_hbm, s_hbm, w_v, dst_v, ex_v, s_v):
    wid = lax.axis_index("s") * NC + lax.axis_index("c")
    pltpu.sync_copy(w_hbm.at[wid], w_v)
    pltpu.sync_copy(dst_hbm.at[wid], dst_v)

    def zero_body(i, carry):
        s_v[pl.ds(i * L, L)] = jnp.zeros((L,), jnp.float32)
        return carry

    lax.fori_loop(0, N_NODES // L, zero_body, 0)

    def body(i, carry):
        e = jnp.exp(w_v[pl.ds(i * L, L)])
        ex_v[pl.ds(i * L, L)] = e
        plsc.addupdate_scatter(s_v, [dst_v[pl.ds(i * L, L)]], e)
        return carry

    lax.fori_loop(0, EPW // L, body, 0)
    pltpu.sync_copy(ex_v, ex_hbm.at[wid])
    pltpu.sync_copy(s_v, s_hbm.at[wid])


def _propagate_body(h_hbm, src_hbm, dst_hbm, ex_hbm, out_hbm,
               src_v, dst_v, ex_v, rows, acc, sem):
    cid = lax.axis_index("c")
    sid = lax.axis_index("s")
    wid = sid * NC + cid
    pltpu.sync_copy(src_hbm.at[wid], src_v)
    pltpu.sync_copy(dst_hbm.at[wid], dst_v)
    pltpu.sync_copy(ex_hbm.at[wid], ex_v)

    def zrow(r, carry):
        for c in range(CPD):
            rows[r, pl.ds(c * L, L)] = jnp.zeros((L,), jnp.float32)
        return carry

    lax.fori_loop(0, K, zrow, 0)
    for kk in range(RPT // K):
        pltpu.sync_copy(rows, acc.at[pl.ds(sid * RPT + kk * K, K)])
    plsc.subcore_barrier()

    def chunk_body(j, carry):
        pltpu.sync_copy(h_hbm.at[src_v.at[j]], rows)

        def row_body(r, c2):
            splat = plsc.load_gather(
                ex_v, [jnp.full((L,), j * K + r, dtype=jnp.int32)])
            for c in range(CPD):
                rows[r, pl.ds(c * L, L)] = rows[r, pl.ds(c * L, L)] * splat
            return c2

        lax.fori_loop(0, K, row_body, 0)
        pltpu.sync_copy(rows, acc.at[dst_v.at[j]], add=True)
        return carry

    lax.fori_loop(0, NCHUNK, chunk_body, 0)
    plsc.subcore_barrier()

    @pl.when(sid < NS - 1)
    def _():
        pltpu.sync_copy(acc.at[pl.ds(sid * RPA, RPA)],
                        out_hbm.at[cid, pl.ds(sid * RPA, RPA)])

    @pl.when(sid == NS - 1)
    def _():
        pltpu.sync_copy(acc.at[pl.ds((NS - 1) * RPA, RPL)],
                        out_hbm.at[cid, pl.ds((NS - 1) * RPA, RPL)])


_softmax_stats = pl.kernel(
    _softmax_stats_body,
    out_type=(
        jax.ShapeDtypeStruct((NW, EPW), jnp.float32),
        jax.ShapeDtypeStruct((NW, N_NODES), jnp.float32),
    ),
    mesh=_mesh,
    scratch_types=[
        pltpu.VMEM((EPW,), jnp.float32),
        pltpu.VMEM((EPW,), jnp.int32),
        pltpu.VMEM((EPW,), jnp.float32),
        pltpu.VMEM((N_NODES,), jnp.float32),
    ],
    compiler_params=pltpu.CompilerParams(needs_layout_passes=False),
)

_propagate = pl.kernel(
    _propagate_body,
    out_type=jax.ShapeDtypeStruct((NC, N_NODES, D), jnp.float32),
    mesh=_mesh,
    scratch_types=[
        pltpu.VMEM((NCHUNK, K), jnp.int32),
        pltpu.VMEM((NCHUNK, K), jnp.int32),
        pltpu.VMEM((EPW,), jnp.float32),
        pltpu.VMEM((K, D), jnp.float32),
        pltpu.VMEM_SHARED((N_NODES, D), jnp.float32),
        pltpu.SemaphoreType.DMA,
    ],
    compiler_params=pltpu.CompilerParams(needs_layout_passes=False),
)


def _combine(parts, s32t):
    nb = 5
    bn = N_NODES // nb

    def body(p_ref, s_ref, o_ref):
        s = jnp.sum(s_ref[...], axis=1)
        r = 1.0 / (s + 1e-16)
        p = p_ref[...]
        o_ref[...] = (p[0] + p[1]) * r[:, None]

    return pl.pallas_call(
        body,
        out_shape=jax.ShapeDtypeStruct((N_NODES, D), jnp.float32),
        grid=(nb,),
        in_specs=[
            pl.BlockSpec((2, bn, D), lambda i: (0, i, 0)),
            pl.BlockSpec((bn, NW), lambda i: (i, 0)),
        ],
        out_specs=pl.BlockSpec((bn, D), lambda i: (i, 0)),
    )(parts, s32t)


def kernel(x, edge_index, edge_weight):
    src3 = edge_index[0].reshape(NW, NCHUNK, K)
    dst3 = edge_index[1].reshape(NW, NCHUNK, K)
    dst2 = edge_index[1].reshape(NW, EPW)
    w2 = edge_weight.reshape(NW, EPW)

    ex2, s32 = _softmax_stats(w2, dst2)
    s32t = s32.T
    h = x
    for _ in range(3):
        parts = _propagate(h, src3, dst3, ex2)
        h = _combine(parts, s32t)
    return h


# speedup vs baseline: 18.0199x; 1.6287x over previous
import functools

import jax
import jax.numpy as jnp
from jax import lax
from jax.experimental import pallas as pl
from jax.experimental.pallas import tpu as pltpu
from jax.experimental.pallas import tpu_sc as plsc

NC = 2
NS = 16
L = 16
NW = NC * NS

N_NODES = 10000
N_EDGES = 320000
D = 128

EPW = N_EDGES // NW
K = 125
NCHUNK = EPW // K
RPT = N_NODES // NS
RPA = 624
RPL = N_NODES - (NS - 1) * RPA
CPD = D // L

_mesh = plsc.VectorSubcoreMesh(core_axis_name="c", subcore_axis_name="s")


def _softmax_stats_body(w_hbm, dst_hbm, ex_hbm, s_hbm, w_v, dst_v, ex_v, s_v):
    wid = lax.axis_index("s") * NC + lax.axis_index("c")
    pltpu.sync_copy(w_hbm.at[wid], w_v)
    pltpu.sync_copy(dst_hbm.at[wid], dst_v)

    def zero_body(i, carry):
        s_v[pl.ds(i * L, L)] = jnp.zeros((L,), jnp.float32)
        return carry

    lax.fori_loop(0, N_NODES // L, zero_body, 0)

    def body(i, carry):
        e = jnp.exp(w_v[pl.ds(i * L, L)])
        ex_v[pl.ds(i * L, L)] = e
        plsc.addupdate_scatter(s_v, [dst_v[pl.ds(i * L, L)]], e)
        return carry

    lax.fori_loop(0, EPW // L, body, 0)
    pltpu.sync_copy(ex_v, ex_hbm.at[wid])
    pltpu.sync_copy(s_v, s_hbm.at[wid])


def _propagate_body(h_hbm, src_hbm, dst_hbm, ex_hbm, out_hbm,
                    src_v, rows_a, rows_b, dst_b2, ex_a, ex_b, acc,
                    sem_a, sem_b, sem_ma, sem_mb):
    cid = lax.axis_index("c")
    sid = lax.axis_index("s")
    wid = sid * NC + cid
    pltpu.sync_copy(src_hbm.at[wid], src_v)

    def zrow(r, carry):
        for c in range(CPD):
            rows_a[r, pl.ds(c * L, L)] = jnp.zeros((L,), jnp.float32)
        return carry

    lax.fori_loop(0, K, zrow, 0)
    for kk in range(RPT // K):
        pltpu.sync_copy(rows_a, acc.at[pl.ds(sid * RPT + kk * K, K)])
    plsc.subcore_barrier()

    def process(j, buf, exr, dstrow):
        def row_body(r, c2):
            splat = plsc.load_gather(ex_ref=exr, indices=None) if False else                 plsc.load_gather(exr, [jnp.full((L,), r, dtype=jnp.int32)])
            for c in range(CPD):
                buf[r, pl.ds(c * L, L)] = buf[r, pl.ds(c * L, L)] * splat
            return c2

        lax.fori_loop(0, K, row_body, 0, unroll=4)
        pltpu.sync_copy(buf, acc.at[dstrow], add=True)

    pltpu.async_copy(dst_hbm.at[wid, 0, :], dst_b2.at[0], sem_ma)
    pltpu.async_copy(ex_hbm.at[wid, 0, :], ex_a, sem_ma)
    pltpu.async_copy(h_hbm.at[src_v.at[0]], rows_a, sem_a)

    def pair_body(i, carry):
        j0 = 2 * i
        pltpu.make_async_copy(h_hbm.at[src_v.at[j0]], rows_a, sem_a).wait()
        pltpu.async_copy(dst_hbm.at[wid, j0 + 1, :], dst_b2.at[1], sem_mb)
        pltpu.async_copy(ex_hbm.at[wid, j0 + 1, :], ex_b, sem_mb)
        pltpu.async_copy(h_hbm.at[src_v.at[j0 + 1]], rows_b, sem_b)
        pltpu.make_async_copy(dst_hbm.at[wid, j0, :], dst_b2.at[0], sem_ma).wait()
        pltpu.make_async_copy(ex_hbm.at[wid, j0, :], ex_a, sem_ma).wait()
        process(j0, rows_a, ex_a, dst_b2.at[0])
        pltpu.make_async_copy(h_hbm.at[src_v.at[j0 + 1]], rows_b, sem_b).wait()

        @pl.when(i < NCHUNK // 2 - 1)
        def _():
            pltpu.async_copy(dst_hbm.at[wid, j0 + 2, :], dst_b2.at[0], sem_ma)
            pltpu.async_copy(ex_hbm.at[wid, j0 + 2, :], ex_a, sem_ma)
            pltpu.async_copy(h_hbm.at[src_v.at[j0 + 2]], rows_a, sem_a)

        pltpu.make_async_copy(dst_hbm.at[wid, j0 + 1, :], dst_b2.at[1], sem_mb).wait()
        pltpu.make_async_copy(ex_hbm.at[wid, j0 + 1, :], ex_b, sem_mb).wait()
        process(j0 + 1, rows_b, ex_b, dst_b2.at[1])
        return carry

    lax.fori_loop(0, NCHUNK // 2, pair_body, 0)
    plsc.subcore_barrier()

    @pl.when(sid < NS - 1)
    def _():
        pltpu.sync_copy(acc.at[pl.ds(sid * RPA, RPA)],
                        out_hbm.at[cid, pl.ds(sid * RPA, RPA)])

    @pl.when(sid == NS - 1)
    def _():
        pltpu.sync_copy(acc.at[pl.ds((NS - 1) * RPA, RPL)],
                        out_hbm.at[cid, pl.ds((NS - 1) * RPA, RPL)])


_softmax_stats = pl.kernel(
    _softmax_stats_body,
    out_type=(
        jax.ShapeDtypeStruct((NW, EPW), jnp.float32),
        jax.ShapeDtypeStruct((NW, N_NODES), jnp.float32),
    ),
    mesh=_mesh,
    scratch_types=[
        pltpu.VMEM((EPW,), jnp.float32),
        pltpu.VMEM((EPW,), jnp.int32),
        pltpu.VMEM((EPW,), jnp.float32),
        pltpu.VMEM((N_NODES,), jnp.float32),
    ],
    compiler_params=pltpu.CompilerParams(needs_layout_passes=False),
)

_propagate = pl.kernel(
    _propagate_body,
    out_type=jax.ShapeDtypeStruct((NC, N_NODES, D), jnp.float32),
    mesh=_mesh,
    scratch_types=[
        pltpu.VMEM((NCHUNK, K), jnp.int32),
        pltpu.VMEM((K, D), jnp.float32),
        pltpu.VMEM((K, D), jnp.float32),
        pltpu.VMEM((2, K), jnp.int32),
        pltpu.VMEM((K,), jnp.float32),
        pltpu.VMEM((K,), jnp.float32),
        pltpu.VMEM_SHARED((N_NODES, D), jnp.float32),
        pltpu.SemaphoreType.DMA,
        pltpu.SemaphoreType.DMA,
        pltpu.SemaphoreType.DMA,
        pltpu.SemaphoreType.DMA,
    ],
    compiler_params=pltpu.CompilerParams(needs_layout_passes=False),
)


def _combine(parts, s32t):
    nb = 5
    bn = N_NODES // nb

    def body(p_ref, s_ref, o_ref):
        s = jnp.sum(s_ref[...], axis=1)
        r = 1.0 / (s + 1e-16)
        p = p_ref[...]
        o_ref[...] = (p[0] + p[1]) * r[:, None]

    return pl.pallas_call(
        body,
        out_shape=jax.ShapeDtypeStruct((N_NODES, D), jnp.float32),
        grid=(nb,),
        in_specs=[
            pl.BlockSpec((2, bn, D), lambda i: (0, i, 0)),
            pl.BlockSpec((bn, NW), lambda i: (i, 0)),
        ],
        out_specs=pl.BlockSpec((bn, D), lambda i: (i, 0)),
    )(parts, s32t)


def kernel(x, edge_index, edge_weight):
    src3 = edge_index[0].reshape(NW, NCHUNK, K)
    dst3 = edge_index[1].reshape(NW, NCHUNK, K)
    dst2 = edge_index[1].reshape(NW, EPW)
    w2 = edge_weight.reshape(NW, EPW)

    ex2, s32 = _softmax_stats(w2, dst2)
    ex3 = ex2.reshape(NW, NCHUNK, K)
    s32t = s32.T
    h = x
    for _ in range(3):
        parts = _propagate(h, src3, dst3, ex3)
        h = _combine(parts, s32t)
    return h


# speedup vs baseline: 20.8925x; 1.1594x over previous
import jax
import jax.numpy as jnp
from jax import lax
from jax.experimental import pallas as pl
from jax.experimental.pallas import tpu as pltpu
from jax.experimental.pallas import tpu_sc as plsc

NC = 2
NS = 16
L = 16
NW = NC * NS

N_NODES = 10000
N_EDGES = 320000
D = 128

EPW = N_EDGES // NW
K = 125
NCHUNK = EPW // K
RPT = N_NODES // NS
RPA = 624
RPL = N_NODES - (NS - 1) * RPA
CPD = D // L

_mesh = plsc.VectorSubcoreMesh(core_axis_name="c", subcore_axis_name="s")


def _softmax_stats_body(w_hbm, dst_hbm, ex_hbm, s_hbm, w_v, dst_v, ex_v, s_v):
    wid = lax.axis_index("s") * NC + lax.axis_index("c")
    pltpu.sync_copy(w_hbm.at[wid], w_v)
    pltpu.sync_copy(dst_hbm.at[wid], dst_v)

    def zero_body(i, carry):
        s_v[pl.ds(i * L, L)] = jnp.zeros((L,), jnp.float32)
        return carry

    lax.fori_loop(0, N_NODES // L, zero_body, 0)

    def body(i, carry):
        e = jnp.exp(w_v[pl.ds(i * L, L)])
        ex_v[pl.ds(i * L, L)] = e
        plsc.addupdate_scatter(s_v, [dst_v[pl.ds(i * L, L)]], e)
        return carry

    lax.fori_loop(0, EPW // L, body, 0)
    pltpu.sync_copy(ex_v, ex_hbm.at[wid])
    pltpu.sync_copy(s_v, s_hbm.at[wid])


def _propagate_body(h_hbm, src_hbm, dst_hbm, ex_hbm, out_hbm,
                    src_v, rows_a, rows_b, dst_b2, ex_a, ex_b, acc,
                    sem_a, sem_b, sem_ma, sem_mb):
    cid = lax.axis_index("c")
    sid = lax.axis_index("s")
    wid = sid * NC + cid
    pltpu.sync_copy(src_hbm.at[wid], src_v)

    def zrow(r, carry):
        for c in range(CPD):
            rows_a[r, pl.ds(c * L, L)] = jnp.zeros((L,), jnp.float32)
        return carry

    lax.fori_loop(0, K, zrow, 0)
    for kk in range(RPT // K):
        pltpu.sync_copy(rows_a, acc.at[pl.ds(sid * RPT + kk * K, K)])
    plsc.subcore_barrier()

    def process(j, buf, exr, dstrow):
        @plsc.parallel_loop(0, K, unroll=4)
        def _(r):
            splat = plsc.load_gather(
                exr, [jnp.full((L,), r, dtype=jnp.int32)])
            for c in range(CPD):
                buf[r, pl.ds(c * L, L)] = buf[r, pl.ds(c * L, L)] * splat

        pltpu.sync_copy(buf, acc.at[dstrow], add=True)

    pltpu.async_copy(dst_hbm.at[wid, 0, :], dst_b2.at[0], sem_ma)
    pltpu.async_copy(ex_hbm.at[wid, 0, :], ex_a, sem_ma)
    pltpu.async_copy(h_hbm.at[src_v.at[0]], rows_a, sem_a)

    def pair_body(i, carry):
        j0 = 2 * i
        pltpu.make_async_copy(h_hbm.at[src_v.at[j0]], rows_a, sem_a).wait()
        pltpu.async_copy(dst_hbm.at[wid, j0 + 1, :], dst_b2.at[1], sem_mb)
        pltpu.async_copy(ex_hbm.at[wid, j0 + 1, :], ex_b, sem_mb)
        pltpu.async_copy(h_hbm.at[src_v.at[j0 + 1]], rows_b, sem_b)
        pltpu.make_async_copy(dst_hbm.at[wid, j0, :], dst_b2.at[0], sem_ma).wait()
        pltpu.make_async_copy(ex_hbm.at[wid, j0, :], ex_a, sem_ma).wait()
        process(j0, rows_a, ex_a, dst_b2.at[0])
        pltpu.make_async_copy(h_hbm.at[src_v.at[j0 + 1]], rows_b, sem_b).wait()

        @pl.when(i < NCHUNK // 2 - 1)
        def _():
            pltpu.async_copy(dst_hbm.at[wid, j0 + 2, :], dst_b2.at[0], sem_ma)
            pltpu.async_copy(ex_hbm.at[wid, j0 + 2, :], ex_a, sem_ma)
            pltpu.async_copy(h_hbm.at[src_v.at[j0 + 2]], rows_a, sem_a)

        pltpu.make_async_copy(dst_hbm.at[wid, j0 + 1, :], dst_b2.at[1], sem_mb).wait()
        pltpu.make_async_copy(ex_hbm.at[wid, j0 + 1, :], ex_b, sem_mb).wait()
        process(j0 + 1, rows_b, ex_b, dst_b2.at[1])
        return carry

    lax.fori_loop(0, NCHUNK // 2, pair_body, 0)
    plsc.subcore_barrier()

    @pl.when(sid < NS - 1)
    def _():
        pltpu.sync_copy(acc.at[pl.ds(sid * RPA, RPA)],
                        out_hbm.at[cid, pl.ds(sid * RPA, RPA)])

    @pl.when(sid == NS - 1)
    def _():
        pltpu.sync_copy(acc.at[pl.ds((NS - 1) * RPA, RPL)],
                        out_hbm.at[cid, pl.ds((NS - 1) * RPA, RPL)])


_softmax_stats = pl.kernel(
    _softmax_stats_body,
    out_type=(
        jax.ShapeDtypeStruct((NW, EPW), jnp.float32),
        jax.ShapeDtypeStruct((NW, N_NODES), jnp.float32),
    ),
    mesh=_mesh,
    scratch_types=[
        pltpu.VMEM((EPW,), jnp.float32),
        pltpu.VMEM((EPW,), jnp.int32),
        pltpu.VMEM((EPW,), jnp.float32),
        pltpu.VMEM((N_NODES,), jnp.float32),
    ],
    compiler_params=pltpu.CompilerParams(needs_layout_passes=False),
)

_propagate = pl.kernel(
    _propagate_body,
    out_type=jax.ShapeDtypeStruct((NC, N_NODES, D), jnp.float32),
    mesh=_mesh,
    scratch_types=[
        pltpu.VMEM((NCHUNK, K), jnp.int32),
        pltpu.VMEM((K, D), jnp.float32),
        pltpu.VMEM((K, D), jnp.float32),
        pltpu.VMEM((2, K), jnp.int32),
        pltpu.VMEM((K,), jnp.float32),
        pltpu.VMEM((K,), jnp.float32),
        pltpu.VMEM_SHARED((N_NODES, D), jnp.float32),
        pltpu.SemaphoreType.DMA,
        pltpu.SemaphoreType.DMA,
        pltpu.SemaphoreType.DMA,
        pltpu.SemaphoreType.DMA,
    ],
    compiler_params=pltpu.CompilerParams(needs_layout_passes=False),
)


def _combine(parts, s32t):
    nb = 5
    bn = N_NODES // nb

    def body(p_ref, s_ref, o_ref):
        s = jnp.sum(s_ref[...], axis=1)
        r = 1.0 / (s + 1e-16)
        p = p_ref[...]
        o_ref[...] = (p[0] + p[1]) * r[:, None]

    return pl.pallas_call(
        body,
        out_shape=jax.ShapeDtypeStruct((N_NODES, D), jnp.float32),
        grid=(nb,),
        in_specs=[
            pl.BlockSpec((2, bn, D), lambda i: (0, i, 0)),
            pl.BlockSpec((bn, NW), lambda i: (i, 0)),
        ],
        out_specs=pl.BlockSpec((bn, D), lambda i: (i, 0)),
    )(parts, s32t)


def kernel(x, edge_index, edge_weight):
    src3 = edge_index[0].reshape(NW, NCHUNK, K)
    dst3 = edge_index[1].reshape(NW, NCHUNK, K)
    dst2 = edge_index[1].reshape(NW, EPW)
    w2 = edge_weight.reshape(NW, EPW)

    ex2, s32 = _softmax_stats(w2, dst2)
    ex3 = ex2.reshape(NW, NCHUNK, K)
    s32t = s32.T
    h = x
    for _ in range(3):
        parts = _propagate(h, src3, dst3, ex3)
        h = _combine(parts, s32t)
    return h
